# baseline (speedup 1.0000x reference)
import jax
import jax.numpy as jnp
from jax import lax
from jax.experimental import pallas as pl
from jax.experimental.pallas import tpu as pltpu

_LANES = 128


def _make_body(B, S, C):
    def body(len_ref, xt_hbm, gamma_ref, beta_ref, out_ref,
             stripes, onehot, sem):
        def row_of(b):
            return lax.rem(len_ref[b] + (S - 1), S)

        lane_ids = lax.broadcasted_iota(jnp.int32, (1, _LANES), 1)
        for b in range(B):
            row = row_of(b)
            lane0 = pl.multiple_of(lax.div(row, _LANES) * _LANES, _LANES)
            pltpu.make_async_copy(
                xt_hbm.at[b, :, pl.ds(lane0, _LANES)], stripes.at[b],
                sem.at[b],
            ).start()
            sub = lax.rem(row, _LANES)
            onehot[b:b + 1] = jnp.where(lane_ids == sub, 1.0, 0.0)

        gam = gamma_ref[...]
        bet = beta_ref[...]
        for b in range(B):
            pltpu.make_async_copy(
                xt_hbm.at[0, :, pl.ds(0, _LANES)], stripes.at[b],
                sem.at[b],
            ).wait()
            picked = lax.dot_general(
                onehot[b:b + 1], stripes[b],
                dimension_numbers=(((1,), (1,)), ((), ())),
                precision=lax.Precision.HIGHEST,
            )
            out_ref[b:b + 1] = (picked * gam + bet)[:, None, :]

    return body


@jax.jit
def kernel(x, length, gamma, beta):
    B, S, C = x.shape
    xt = x.transpose(0, 2, 1)
    out = pl.pallas_call(
        _make_body(B, S, C),
        grid_spec=pltpu.PrefetchScalarGridSpec(
            num_scalar_prefetch=1,
            grid=(1,),
            in_specs=[
                pl.BlockSpec(memory_space=pl.ANY),
                pl.BlockSpec((1, C), lambda i, len_ref: (0, 0)),
                pl.BlockSpec((1, C), lambda i, len_ref: (0, 0)),
            ],
            out_specs=pl.BlockSpec((B, 1, C), lambda i, len_ref: (0, 0, 0)),
            scratch_shapes=[
                pltpu.VMEM((B, C, _LANES), jnp.float32),
                pltpu.VMEM((B, _LANES), jnp.float32),
                pltpu.SemaphoreType.DMA((B,)),
            ],
        ),
        out_shape=jax.ShapeDtypeStruct((B, 1, C), jnp.float32),
    )(length.astype(jnp.int32), xt, gamma, beta)
    return out


# speedup vs baseline: 1.5835x; 1.5835x over previous
import jax
import jax.numpy as jnp
from jax import lax
from jax.experimental import pallas as pl
from jax.experimental.pallas import tpu as pltpu

_LANES = 128


def _make_body(B, S, C):
    def body(len_ref, xt_hbm, gamma_ref, beta_ref, out_ref,
             stripes, onehot, sem):
        def row_of(b):
            return lax.rem(len_ref[b] + (S - 1), S)

        def fire(b, _):
            row = row_of(b)
            lane0 = pl.multiple_of(lax.div(row, _LANES) * _LANES, _LANES)
            pltpu.make_async_copy(
                xt_hbm.at[b, :, pl.ds(lane0, _LANES)], stripes.at[b],
                sem.at[b],
            ).start()
            sub = lax.rem(row, _LANES)
            lane_ids = lax.broadcasted_iota(jnp.int32, (1, _LANES), 1)
            onehot[pl.ds(b, 1)] = jnp.where(lane_ids == sub, 1.0, 0.0)
            return 0
        lax.fori_loop(0, B, fire, 0)

        def wait(b, _):
            pltpu.make_async_copy(
                xt_hbm.at[0, :, pl.ds(0, _LANES)], stripes.at[b],
                sem.at[b],
            ).wait()
            return 0

        G = 8
        gam = gamma_ref[...]
        bet = beta_ref[...]
        for g in range(0, B, G):
            lax.fori_loop(g, g + G, wait, 0)
            grp = stripes[g:g + G]
            oh = onehot[g:g + G]
            picked = jnp.sum(grp * oh[:, None, :], axis=2)
            out_ref[g:g + G] = (picked * gam + bet)[:, None, :]

    return body


@jax.jit
def kernel(x, length, gamma, beta):
    B, S, C = x.shape
    xt = x.transpose(0, 2, 1)
    out = pl.pallas_call(
        _make_body(B, S, C),
        grid_spec=pltpu.PrefetchScalarGridSpec(
            num_scalar_prefetch=1,
            grid=(1,),
            in_specs=[
                pl.BlockSpec(memory_space=pl.ANY),
                pl.BlockSpec((1, C), lambda i, len_ref: (0, 0)),
                pl.BlockSpec((1, C), lambda i, len_ref: (0, 0)),
            ],
            out_specs=pl.BlockSpec((B, 1, C), lambda i, len_ref: (0, 0, 0)),
            scratch_shapes=[
                pltpu.VMEM((B, C, _LANES), jnp.float32),
                pltpu.VMEM((B, _LANES), jnp.float32),
                pltpu.SemaphoreType.DMA((B,)),
            ],
        ),
        out_shape=jax.ShapeDtypeStruct((B, 1, C), jnp.float32),
    )(length.astype(jnp.int32), xt, gamma, beta)
    return out
